# Initial kernel scaffold; baseline (speedup 1.0000x reference)
import jax
import jax.numpy as jnp
from jax.experimental import pallas as pl


def kernel(seq, tables):
    raise NotImplementedError("write your pallas kernel here")


# speedup vs baseline: 5.5376x; 5.5376x over previous
import jax
import jax.numpy as jnp
from jax import lax
from jax.experimental import pallas as pl
from jax.experimental.pallas import tpu as pltpu
from jax.experimental.pallas import tpu_sc as plsc

N_VOCAB = 100000
N_HEADS = 8
N_FEATURES = 64
BATCH = 4096
HIST = 50

GROUP_B = 8
ROWS = GROUP_B * HIST
CHUNKS = 5
CHUNK = ROWS // CHUNKS
VECS = ROWS // 16
VPC = VECS // CHUNKS


def _make_kernel():
    info = plsc.get_sparse_core_info()
    nc, ns = info.num_cores, info.num_subcores
    nw = nc * ns
    n_groups = BATCH // GROUP_B
    gpt = n_groups // nw

    mesh = plsc.VectorSubcoreMesh(core_axis_name="c", subcore_axis_name="s")

    def body(seq_hbm, tab_hbm, out_hbm, seq_v, idx_v, rows_v,
             gsem, ssem0, ssem1):
        wid = lax.axis_index("s") * nc + lax.axis_index("c")

        def drain_stores(slot, ssem, b0, h):
            for k in range(GROUP_B):
                pltpu.make_async_copy(
                    rows_v.at[slot, pl.ds(k * HIST, HIST)],
                    out_hbm.at[b0 + k, h], ssem).wait()

        def group_body(gi, carry):
            g = wid * gpt + gi
            b0 = g * GROUP_B
            pltpu.sync_copy(seq_hbm.at[pl.ds(g * ROWS, ROWS)], seq_v)
            for h in range(N_HEADS):
                slot = h % 2
                ssem = ssem0 if slot == 0 else ssem1
                if h >= 2:
                    drain_stores(slot, ssem, b0, h)
                else:
                    @pl.when(gi > 0)
                    def _():
                        drain_stores(slot, ssem, b0, h)
                off = jnp.int32(h * N_VOCAB)
                for p in range(VECS):
                    idx_v[slot, p // VPC, pl.ds((p % VPC) * 16, 16)] = (
                        seq_v[pl.ds(p * 16, 16)] + off)
                gathers = [
                    pltpu.async_copy(
                        tab_hbm.at[idx_v.at[slot, j]],
                        rows_v.at[slot, pl.ds(j * CHUNK, CHUNK)], gsem)
                    for j in range(CHUNKS)
                ]
                for c in gathers:
                    c.wait()
                for k in range(GROUP_B):
                    pltpu.async_copy(
                        rows_v.at[slot, pl.ds(k * HIST, HIST)],
                        out_hbm.at[b0 + k, h], ssem)
            return carry

        lax.fori_loop(0, gpt, group_body, 0)
        b_last = (wid * gpt + gpt - 1) * GROUP_B
        drain_stores(0, ssem0, b_last, N_HEADS - 2)
        drain_stores(1, ssem1, b_last, N_HEADS - 1)

    return pl.kernel(
        body,
        out_type=jax.ShapeDtypeStruct(
            (BATCH, N_HEADS, HIST, N_FEATURES), jnp.float32),
        mesh=mesh,
        scratch_types=[
            pltpu.VMEM((ROWS,), jnp.int32),
            pltpu.VMEM((2, CHUNKS, CHUNK), jnp.int32),
            pltpu.VMEM((2, ROWS, N_FEATURES), jnp.float32),
            pltpu.SemaphoreType.DMA,
            pltpu.SemaphoreType.DMA,
            pltpu.SemaphoreType.DMA,
        ],
        compiler_params=pltpu.CompilerParams(use_tc_tiling_on_sc=False),
    )


def kernel(seq, tables):
    seq_flat = seq.reshape(-1).astype(jnp.int32)
    tab_flat = tables.reshape(N_HEADS * N_VOCAB, N_FEATURES)
    return _make_kernel()(seq_flat, tab_flat)


# speedup vs baseline: 5.5990x; 1.0111x over previous
import jax
import jax.numpy as jnp
from jax import lax
from jax.experimental import pallas as pl
from jax.experimental.pallas import tpu as pltpu
from jax.experimental.pallas import tpu_sc as plsc

N_VOCAB = 100000
N_HEADS = 8
N_FEATURES = 64
BATCH = 4096
HIST = 50

GROUP_B = 8
ROWS = GROUP_B * HIST
CHUNKS = 5
CHUNK = ROWS // CHUNKS
VECS = ROWS // 16
VPC = VECS // CHUNKS
NSLOT = 4


def _make_kernel():
    info = plsc.get_sparse_core_info()
    nc, ns = info.num_cores, info.num_subcores
    nw = nc * ns
    n_groups = BATCH // GROUP_B
    gpt = n_groups // nw

    mesh = plsc.VectorSubcoreMesh(core_axis_name="c", subcore_axis_name="s")

    def body(seq_hbm, tab_hbm, out_hbm, seq_v, idx_v, rows_v, gsem, ssem):
        wid = lax.axis_index("s") * nc + lax.axis_index("c")

        def drain(slot, sem):
            pltpu.make_async_copy(
                tab_hbm.at[pl.ds(0, ROWS)], rows_v.at[slot],
                sem.at[slot]).wait()

        def build_and_fire(slot, b0, h):
            off = jnp.int32(h * N_VOCAB)
            for p in range(VECS):
                idx_v[slot, p // VPC, pl.ds((p % VPC) * 16, 16)] = (
                    seq_v[pl.ds(p * 16, 16)] + off)
            for j in range(CHUNKS):
                pltpu.async_copy(
                    tab_hbm.at[idx_v.at[slot, j]],
                    rows_v.at[slot, pl.ds(j * CHUNK, CHUNK)],
                    gsem.at[slot])

        def fire_stores(slot, b0, h):
            for k in range(GROUP_B):
                pltpu.async_copy(
                    rows_v.at[slot, pl.ds(k * HIST, HIST)],
                    out_hbm.at[b0 + k, h], ssem.at[slot])

        def group_body(gi, carry):
            g = wid * gpt + gi
            b0 = g * GROUP_B
            pltpu.sync_copy(seq_hbm.at[pl.ds(g * ROWS, ROWS)], seq_v)
            for h in range(N_HEADS):
                slot = h % NSLOT
                if h >= NSLOT:
                    drain(slot, ssem)
                else:
                    @pl.when(gi > 0)
                    def _():
                        drain(slot, ssem)
                build_and_fire(slot, b0, h)
                pslot = (h - 1) % NSLOT
                if h >= 1:
                    drain(pslot, gsem)
                    fire_stores(pslot, b0, h - 1)
                else:
                    @pl.when(gi > 0)
                    def _():
                        drain(pslot, gsem)
                        fire_stores(pslot, b0 - GROUP_B, N_HEADS - 1)
            return carry

        lax.fori_loop(0, gpt, group_body, 0)
        b_last = (wid * gpt + gpt - 1) * GROUP_B
        last_slot = (N_HEADS - 1) % NSLOT
        drain(last_slot, gsem)
        fire_stores(last_slot, b_last, N_HEADS - 1)
        for slot in range(NSLOT):
            drain(slot, ssem)

    return pl.kernel(
        body,
        out_type=jax.ShapeDtypeStruct(
            (BATCH, N_HEADS, HIST, N_FEATURES), jnp.float32),
        mesh=mesh,
        scratch_types=[
            pltpu.VMEM((ROWS,), jnp.int32),
            pltpu.VMEM((NSLOT, CHUNKS, CHUNK), jnp.int32),
            pltpu.VMEM((NSLOT, ROWS, N_FEATURES), jnp.float32),
            pltpu.SemaphoreType.DMA((NSLOT,)),
            pltpu.SemaphoreType.DMA((NSLOT,)),
        ],
        compiler_params=pltpu.CompilerParams(use_tc_tiling_on_sc=False),
    )


def kernel(seq, tables):
    seq_flat = seq.reshape(-1).astype(jnp.int32)
    tab_flat = tables.reshape(N_HEADS * N_VOCAB, N_FEATURES)
    return _make_kernel()(seq_flat, tab_flat)
